# baseline (speedup 1.0000x reference)
import functools
import math

import jax
import jax.numpy as jnp
from jax import lax
from jax.experimental import pallas as pl
from jax.experimental.pallas import tpu as pltpu
from jax.experimental.pallas import tpu_sc as plsc

LA = 9
WA = 1 << LA
TBA = 256
TBB = 256


def _phase_a_body(x_ref, xh_ref, yh_ref, y_ref, node_ref):
    x = x_ref[...]
    s = lax.dot_general(x, xh_ref[...], (((1,), (1,)), ((), ())),
                        preferred_element_type=jnp.float32,
                        precision=lax.Precision.HIGHEST)
    iota = lax.broadcasted_iota(jnp.int32, (TBA, WA), 1)
    node = jnp.zeros((TBA, 1), jnp.int32)
    acc = jnp.zeros((TBA, WA), jnp.float32)
    for _ in range(LA):
        onehot = iota == node
        lam = jnp.sum(jnp.where(onehot, s, 0.0), axis=1, keepdims=True)
        acc = acc + jnp.where(onehot, lam, 0.0)
        node = 2 * node + 1 + (lam > 0.0).astype(jnp.int32)
    y_ref[...] = lax.dot_general(acc, yh_ref[...], (((1,), (0,)), ((), ())),
                                 preferred_element_type=jnp.float32)
    node_ref[...] = node.reshape(1, 1, TBA)


def _phase_a(x, xh, yh):
    b, f = x.shape
    grid = (b // TBA,)
    return pl.pallas_call(
        _phase_a_body,
        grid=grid,
        in_specs=[
            pl.BlockSpec((TBA, f), lambda t: (t, 0)),
            pl.BlockSpec((WA, f), lambda t: (0, 0)),
            pl.BlockSpec((WA, f), lambda t: (0, 0)),
        ],
        out_specs=[
            pl.BlockSpec((TBA, f), lambda t: (t, 0)),
            pl.BlockSpec((1, 1, TBA), lambda t: (t, 0, 0)),
        ],
        out_shape=[
            jax.ShapeDtypeStruct((b, f), jnp.float32),
            jax.ShapeDtypeStruct((b // TBA, 1, TBA), jnp.int32),
        ],
    )(x, xh, yh)


def _upd_body(x_ref, xn_ref, yn_ref, yin_ref, nin_ref, yout_ref, nout_ref):
    lam = jnp.sum(x_ref[...] * xn_ref[...], axis=1, keepdims=True)
    yout_ref[...] = yin_ref[...] + lam * yn_ref[...]
    nout_ref[...] = (2 * nin_ref[...] + 1
                     + (lam > 0.0).astype(jnp.int32).reshape(1, 1, TBB))


def _update(x, xn, yn, yin, nin):
    b, f = x.shape
    grid = (b // TBB,)
    row = pl.BlockSpec((TBB, f), lambda t: (t, 0))
    nspec = pl.BlockSpec((1, 1, TBB), lambda t: (t, 0, 0))
    return pl.pallas_call(
        _upd_body,
        grid=grid,
        in_specs=[row, row, row, row, nspec],
        out_specs=[row, nspec],
        out_shape=[
            jax.ShapeDtypeStruct((b, f), jnp.float32),
            jax.ShapeDtypeStruct((b // TBB, 1, TBB), jnp.int32),
        ],
    )(x, xn, yn, yin, nin)


def _sc_phase_b(xtab, ytab, x, yin, node):
    b, f = x.shape
    info = plsc.get_sparse_core_info()
    nc, ns = info.num_cores, info.num_subcores
    nw = nc * ns
    bpw = b // nw
    nck = f // 16
    mesh = plsc.VectorSubcoreMesh(core_axis_name="c", subcore_axis_name="s")

    row_f32 = pltpu.VMEM((1, f), jnp.float32)

    @functools.partial(
        pl.kernel,
        mesh=mesh,
        out_type=jax.ShapeDtypeStruct((b, f), jnp.float32),
        scratch_types=[
            pltpu.VMEM((bpw,), jnp.int32),
            pltpu.VMEM((bpw, 8), jnp.int32),
            pltpu.VMEM((4, 16), jnp.int32),
        ] + [row_f32] * 16 + [pltpu.SemaphoreType.DMA] * 8,
    )
    def k(xt_hbm, yt_hbm, x_hbm, yin_hbm, idx_hbm, idx2_hbm, yout_hbm,
          idx_v, idx2_v, nodebuf,
          xb0, xb1, xb2, xb3, yb0, yb1, yb2, yb3,
          xn0, xn1, xn2, xn3, yn0, yn1, yn2, yn3,
          si0, si1, si2, si3, so0, so1, so2, so3):
        xbs = (xb0, xb1, xb2, xb3)
        ybs = (yb0, yb1, yb2, yb3)
        xns = (xn0, xn1, xn2, xn3)
        yns = (yn0, yn1, yn2, yn3)
        sin = (si0, si1, si2, si3)
        sout = (so0, so1, so2, so3)
        wid = lax.axis_index("s") * nc + lax.axis_index("c")
        base = wid * bpw
        pltpu.sync_copy(idx_hbm.at[pl.ds(base, bpw)], idx_v)
        pltpu.sync_copy(idx2_hbm.at[pl.ds(base, bpw)], idx2_v)

        def vgather(vec, iv):
            dn = lax.GatherDimensionNumbers(
                offset_dims=(), collapsed_slice_dims=(0,),
                start_index_map=(0,))
            return lax.gather(vec, iv[:, None], dn, slice_sizes=(1,),
                              mode=lax.GatherScatterMode.PROMISE_IN_BOUNDS)

        def issue_s0(tok, slot):
            s = sin[slot]
            pltpu.async_copy(x_hbm.at[pl.ds(base + tok, 1)], xbs[slot], s)
            pltpu.async_copy(yin_hbm.at[pl.ds(base + tok, 1)], ybs[slot], s)
            idxs = idx2_v.at[tok, pl.ds(0, 1)]
            pltpu.async_copy(xt_hbm.at[idxs], xns[slot], s)
            pltpu.async_copy(yt_hbm.at[idxs], yns[slot], s)

        def drain_in(slot, nrows):
            for _ in range(nrows):
                pltpu.make_async_copy(
                    xt_hbm.at[pl.ds(0, 1)], xns[slot], sin[slot]).wait()

        def rowdot(slot):
            xr, xnr = xbs[slot], xns[slot]

            def body(i, acc):
                return acc + (xr[0, pl.ds(i * 16, 16)]
                              * xnr[0, pl.ds(i * 16, 16)])
            acc = lax.fori_loop(0, nck, body, jnp.zeros((16,), jnp.float32))
            lane = lax.broadcasted_iota(jnp.int32, (16,), 0)
            for kbit in (1, 2, 4, 8):
                acc = acc + vgather(acc, lane ^ kbit)
            return acc

        def axpy(lam, slot):
            yr, ynr = ybs[slot], yns[slot]

            def body(i, c):
                yv = yr[0, pl.ds(i * 16, 16)] + lam * ynr[0, pl.ds(i * 16, 16)]
                yr[0, pl.ds(i * 16, 16)] = yv
                return c
            lax.fori_loop(0, nck, body, 0)

        def next_node(nvec_f, lam):
            bf = jnp.maximum(jnp.sign(lam), 0.0)
            return 2.0 * nvec_f + 1.0 + bf

        def issue_gather(slot):
            s = sin[slot]
            idxs = nodebuf.at[slot, pl.ds(0, 1)]
            pltpu.async_copy(xt_hbm.at[idxs], xns[slot], s)
            pltpu.async_copy(yt_hbm.at[idxs], yns[slot], s)

        issue_s0(0, 0)

        def step4(t4, carry):
            t0 = t4 * 4
            for dt in range(4):
                t = t0 + dt
                u = t + 1
                su = (dt + 1) % 4

                @pl.when(u < bpw)
                def _(u=u, su=su):
                    @pl.when(u >= 4)
                    def _():
                        pltpu.make_async_copy(
                            ybs[su],
                            yout_hbm.at[pl.ds(base + u - 4, 1)],
                            sout[su]).wait()
                    issue_s0(u, su)

                sa = dt % 4

                @pl.when(t < bpw)
                def _(a=t, sa=sa):
                    drain_in(sa, 4)
                    lam = rowdot(sa)
                    axpy(lam, sa)
                    v = idx_v[pl.ds((a // 16) * 16, 16)]
                    iv = jnp.full((16,), a % 16, jnp.int32)
                    n9 = vgather(v, iv)
                    n10 = next_node(n9.astype(jnp.float32), lam)
                    nodebuf[sa, pl.ds(0, 16)] = n10.astype(jnp.int32)
                    issue_gather(sa)

                bt = t - 1
                sb = (dt + 3) % 4

                @pl.when((bt >= 0) & (bt < bpw))
                def _(sb=sb):
                    drain_in(sb, 2)
                    lam = rowdot(sb)
                    axpy(lam, sb)
                    n10 = nodebuf[sb, pl.ds(0, 16)]
                    n11 = next_node(n10.astype(jnp.float32), lam)
                    nodebuf[sb, pl.ds(0, 16)] = n11.astype(jnp.int32)
                    issue_gather(sb)

                ct = t - 2
                sc = (dt + 2) % 4

                @pl.when((ct >= 0) & (ct < bpw))
                def _(ct=ct, sc=sc):
                    drain_in(sc, 2)
                    lam = rowdot(sc)
                    axpy(lam, sc)
                    pltpu.async_copy(ybs[sc],
                                     yout_hbm.at[pl.ds(base + ct, 1)],
                                     sout[sc])
            return carry

        lax.fori_loop(0, (bpw + 2 + 3) // 4, step4, 0)

        for kk in range(4):
            tok = bpw - 4 + kk
            slot = tok % 4
            pltpu.make_async_copy(
                ybs[slot],
                yout_hbm.at[pl.ds(base + tok, 1)], sout[slot]).wait()

    node2 = jnp.broadcast_to(node[:, None], (b, 8))
    return k(xtab, ytab, x, yin, node, node2)


def kernel(oldx, X, Y):
    f = X.shape[-1]
    x = oldx.reshape(-1, f)
    b = x.shape[0]
    y, node3 = _phase_a(x, X[:WA], Y[:WA])
    y = _sc_phase_b(X, Y, x, y, node3.reshape(b))
    return y.reshape(oldx.shape)


# speedup vs baseline: 1.2162x; 1.2162x over previous
import functools
import math

import jax
import jax.numpy as jnp
from jax import lax
from jax.experimental import pallas as pl
from jax.experimental.pallas import tpu as pltpu
from jax.experimental.pallas import tpu_sc as plsc

LA = 9
WA = 1 << LA
TBA = 256
TBB = 256


def _phase_a_body(x_ref, xh_ref, yh_ref, y_ref, node_ref):
    x = x_ref[...]
    s = lax.dot_general(x, xh_ref[...], (((1,), (1,)), ((), ())),
                        preferred_element_type=jnp.float32,
                        precision=lax.Precision.HIGHEST)
    iota = lax.broadcasted_iota(jnp.int32, (TBA, WA), 1)
    node = jnp.zeros((TBA, 1), jnp.int32)
    acc = jnp.zeros((TBA, WA), jnp.float32)
    for _ in range(LA):
        onehot = iota == node
        lam = jnp.sum(jnp.where(onehot, s, 0.0), axis=1, keepdims=True)
        acc = acc + jnp.where(onehot, lam, 0.0)
        node = 2 * node + 1 + (lam > 0.0).astype(jnp.int32)
    y_ref[...] = lax.dot_general(acc, yh_ref[...], (((1,), (0,)), ((), ())),
                                 preferred_element_type=jnp.float32)
    node_ref[...] = node.reshape(1, 1, TBA)


def _phase_a(x, xh, yh):
    b, f = x.shape
    grid = (b // TBA,)
    return pl.pallas_call(
        _phase_a_body,
        grid=grid,
        in_specs=[
            pl.BlockSpec((TBA, f), lambda t: (t, 0)),
            pl.BlockSpec((WA, f), lambda t: (0, 0)),
            pl.BlockSpec((WA, f), lambda t: (0, 0)),
        ],
        out_specs=[
            pl.BlockSpec((TBA, f), lambda t: (t, 0)),
            pl.BlockSpec((1, 1, TBA), lambda t: (t, 0, 0)),
        ],
        out_shape=[
            jax.ShapeDtypeStruct((b, f), jnp.float32),
            jax.ShapeDtypeStruct((b // TBA, 1, TBA), jnp.int32),
        ],
    )(x, xh, yh)


def _upd_body(x_ref, xn_ref, yn_ref, yin_ref, nin_ref, yout_ref, nout_ref):
    lam = jnp.sum(x_ref[...] * xn_ref[...], axis=1, keepdims=True)
    yout_ref[...] = yin_ref[...] + lam * yn_ref[...]
    nout_ref[...] = (2 * nin_ref[...] + 1
                     + (lam > 0.0).astype(jnp.int32).reshape(1, 1, TBB))


def _update(x, xn, yn, yin, nin):
    b, f = x.shape
    grid = (b // TBB,)
    row = pl.BlockSpec((TBB, f), lambda t: (t, 0))
    nspec = pl.BlockSpec((1, 1, TBB), lambda t: (t, 0, 0))
    return pl.pallas_call(
        _upd_body,
        grid=grid,
        in_specs=[row, row, row, row, nspec],
        out_specs=[row, nspec],
        out_shape=[
            jax.ShapeDtypeStruct((b, f), jnp.float32),
            jax.ShapeDtypeStruct((b // TBB, 1, TBB), jnp.int32),
        ],
    )(x, xn, yn, yin, nin)


def _sc_phase_b(xtab, ytab, x, yin, node):
    b, f = x.shape
    info = plsc.get_sparse_core_info()
    nc, ns = info.num_cores, info.num_subcores
    nw = nc * ns
    bpw = b // nw
    nck = f // 16
    mesh = plsc.VectorSubcoreMesh(core_axis_name="c", subcore_axis_name="s")

    row_f32 = pltpu.VMEM((1, f), jnp.float32)

    @functools.partial(
        pl.kernel,
        mesh=mesh,
        out_type=jax.ShapeDtypeStruct((b, f), jnp.float32),
        scratch_types=[
            pltpu.VMEM((bpw,), jnp.int32),
            pltpu.VMEM((bpw, 8), jnp.int32),
            pltpu.VMEM((4, 16), jnp.int32),
        ] + [row_f32] * 16 + [pltpu.SemaphoreType.DMA] * 8,
    )
    def k(xt_hbm, yt_hbm, x_hbm, yin_hbm, idx_hbm, idx2_hbm, yout_hbm,
          idx_v, idx2_v, nodebuf,
          xb0, xb1, xb2, xb3, yb0, yb1, yb2, yb3,
          xn0, xn1, xn2, xn3, yn0, yn1, yn2, yn3,
          si0, si1, si2, si3, so0, so1, so2, so3):
        xbs = (xb0, xb1, xb2, xb3)
        ybs = (yb0, yb1, yb2, yb3)
        xns = (xn0, xn1, xn2, xn3)
        yns = (yn0, yn1, yn2, yn3)
        sin = (si0, si1, si2, si3)
        sout = (so0, so1, so2, so3)
        wid = lax.axis_index("s") * nc + lax.axis_index("c")
        base = wid * bpw
        pltpu.sync_copy(idx_hbm.at[pl.ds(base, bpw)], idx_v)
        pltpu.sync_copy(idx2_hbm.at[pl.ds(base, bpw)], idx2_v)

        def vgather(vec, iv):
            dn = lax.GatherDimensionNumbers(
                offset_dims=(), collapsed_slice_dims=(0,),
                start_index_map=(0,))
            return lax.gather(vec, iv[:, None], dn, slice_sizes=(1,),
                              mode=lax.GatherScatterMode.PROMISE_IN_BOUNDS)

        def issue_s0(tok, slot):
            s = sin[slot]
            pltpu.async_copy(x_hbm.at[pl.ds(base + tok, 1)], xbs[slot], s)
            pltpu.async_copy(yin_hbm.at[pl.ds(base + tok, 1)], ybs[slot], s)
            idxs = idx2_v.at[tok, pl.ds(0, 1)]
            pltpu.async_copy(xt_hbm.at[idxs], xns[slot], s)
            pltpu.async_copy(yt_hbm.at[idxs], yns[slot], s)

        def drain_in(slot, nrows):
            for _ in range(nrows):
                pltpu.make_async_copy(
                    xt_hbm.at[pl.ds(0, 1)], xns[slot], sin[slot]).wait()

        def rowdot(slot):
            xr, xnr = xbs[slot], xns[slot]

            def body(i, acc):
                return acc + (xr[0, pl.ds(i * 16, 16)]
                              * xnr[0, pl.ds(i * 16, 16)])
            acc = lax.fori_loop(0, nck, body,
                                jnp.zeros((16,), jnp.float32), unroll=16)
            lane = lax.broadcasted_iota(jnp.int32, (16,), 0)
            for kbit in (1, 2, 4, 8):
                acc = acc + vgather(acc, lane ^ kbit)
            return acc

        def axpy(lam, slot):
            yr, ynr = ybs[slot], yns[slot]

            def body(i, c):
                yv = yr[0, pl.ds(i * 16, 16)] + lam * ynr[0, pl.ds(i * 16, 16)]
                yr[0, pl.ds(i * 16, 16)] = yv
                return c
            lax.fori_loop(0, nck, body, 0, unroll=16)

        def next_node(nvec_f, lam):
            bf = jnp.maximum(jnp.sign(lam), 0.0)
            return 2.0 * nvec_f + 1.0 + bf

        def issue_gather(slot):
            s = sin[slot]
            idxs = nodebuf.at[slot, pl.ds(0, 1)]
            pltpu.async_copy(xt_hbm.at[idxs], xns[slot], s)
            pltpu.async_copy(yt_hbm.at[idxs], yns[slot], s)

        issue_s0(0, 0)

        def step4(t4, carry):
            t0 = t4 * 4
            for dt in range(4):
                t = t0 + dt
                u = t + 1
                su = (dt + 1) % 4

                @pl.when(u < bpw)
                def _(u=u, su=su):
                    @pl.when(u >= 4)
                    def _():
                        pltpu.make_async_copy(
                            ybs[su],
                            yout_hbm.at[pl.ds(base + u - 4, 1)],
                            sout[su]).wait()
                    issue_s0(u, su)

                sa = dt % 4

                @pl.when(t < bpw)
                def _(a=t, sa=sa):
                    drain_in(sa, 4)
                    lam = rowdot(sa)
                    axpy(lam, sa)
                    v = idx_v[pl.ds((a // 16) * 16, 16)]
                    iv = jnp.full((16,), a % 16, jnp.int32)
                    n9 = vgather(v, iv)
                    n10 = next_node(n9.astype(jnp.float32), lam)
                    nodebuf[sa, pl.ds(0, 16)] = n10.astype(jnp.int32)
                    issue_gather(sa)

                bt = t - 1
                sb = (dt + 3) % 4

                @pl.when((bt >= 0) & (bt < bpw))
                def _(sb=sb):
                    drain_in(sb, 2)
                    lam = rowdot(sb)
                    axpy(lam, sb)
                    n10 = nodebuf[sb, pl.ds(0, 16)]
                    n11 = next_node(n10.astype(jnp.float32), lam)
                    nodebuf[sb, pl.ds(0, 16)] = n11.astype(jnp.int32)
                    issue_gather(sb)

                ct = t - 2
                sc = (dt + 2) % 4

                @pl.when((ct >= 0) & (ct < bpw))
                def _(ct=ct, sc=sc):
                    drain_in(sc, 2)
                    lam = rowdot(sc)
                    axpy(lam, sc)
                    pltpu.async_copy(ybs[sc],
                                     yout_hbm.at[pl.ds(base + ct, 1)],
                                     sout[sc])
            return carry

        lax.fori_loop(0, (bpw + 2 + 3) // 4, step4, 0)

        for kk in range(4):
            tok = bpw - 4 + kk
            slot = tok % 4
            pltpu.make_async_copy(
                ybs[slot],
                yout_hbm.at[pl.ds(base + tok, 1)], sout[slot]).wait()

    node2 = jnp.broadcast_to(node[:, None], (b, 8))
    return k(xtab, ytab, x, yin, node, node2)


def kernel(oldx, X, Y):
    f = X.shape[-1]
    x = oldx.reshape(-1, f)
    b = x.shape[0]
    y, node3 = _phase_a(x, X[:WA], Y[:WA])
    y = _sc_phase_b(X, Y, x, y, node3.reshape(b))
    return y.reshape(oldx.shape)
